# Initial kernel scaffold; baseline (speedup 1.0000x reference)
import jax
import jax.numpy as jnp
from jax.experimental import pallas as pl


def kernel(values, prefix_sum):
    raise NotImplementedError("write your pallas kernel here")


# speedup vs baseline: 5.8482x; 5.8482x over previous
import functools

import jax
import jax.numpy as jnp
from jax import lax
from jax.experimental import pallas as pl
from jax.experimental.pallas import tpu as pltpu
from jax.experimental.pallas import tpu_sc as plsc

TOTAL = 32768
NSEG = 16
NTILES = 16
CHUNK = TOTAL // NTILES
LANES = 16
VREGS = CHUNK // LANES

import numpy as np

MINF = np.float32(float("-inf"))
BIG = np.int32(2147483647)


def _body(values_hbm, starts_hbm, ends_hbm, out_hbm,
          vals_v, starts_v, ends_v, my_vals_v, my_idxs_v,
          shared_vals, shared_idxs, merge_vals_v, merge_idxs_v, out_v):
    tid = lax.axis_index("s")
    base = tid * CHUNK

    pltpu.sync_copy(values_hbm.at[pl.ds(base, CHUNK)], vals_v)
    pltpu.sync_copy(starts_hbm, starts_v)
    pltpu.sync_copy(ends_hbm, ends_v)

    lane = lax.iota(jnp.int32, LANES)
    starts_vec = starts_v[...]
    ends_vec = ends_v[...]
    my_vals = jnp.full((LANES,), MINF, jnp.float32)
    my_idxs = jnp.full((LANES,), BIG, jnp.int32)

    for s in range(NSEG):
        lo = starts_vec[s]
        hi = ends_vec[s]
        n0 = jnp.clip(lo - base, 0, CHUNK)
        n1 = jnp.clip(hi - base, 0, CHUNK)
        i0 = n0 >> 4
        i1 = jnp.maximum(i0, (n1 + (LANES - 1)) >> 4)

        def seg_step(i, carry):
            bv, bi = carry
            off = i * LANES
            pos = base + off + lane
            v = vals_v[pl.ds(off, LANES)]
            m = (pos >= lo) & (pos < hi)
            vm = jnp.where(m, v, MINF)
            upd = vm > bv
            bv = jnp.where(upd, vm, bv)
            bi = jnp.where(upd, pos, bi)
            return bv, bi

        bv0 = jnp.full((LANES,), MINF, jnp.float32)
        bi0 = jnp.full((LANES,), BIG, jnp.int32)
        bv, bi = lax.fori_loop(i0, i1, seg_step, (bv0, bi0))

        mx = jnp.max(bv)
        mi = jnp.min(jnp.where(bv == mx, bi, BIG))
        my_vals = jnp.where(lane == s, mx, my_vals)
        my_idxs = jnp.where(lane == s, mi, my_idxs)

    my_vals_v[...] = my_vals
    my_idxs_v[...] = my_idxs
    pltpu.sync_copy(my_vals_v, shared_vals.at[pl.ds(tid * LANES, LANES)])
    pltpu.sync_copy(my_idxs_v, shared_idxs.at[pl.ds(tid * LANES, LANES)])
    plsc.subcore_barrier()

    @pl.when(tid == 0)
    def _merge():
        pltpu.sync_copy(shared_vals, merge_vals_v)
        pltpu.sync_copy(shared_idxs, merge_idxs_v)
        acc_v = merge_vals_v[pl.ds(0, LANES)]
        acc_i = merge_idxs_v[pl.ds(0, LANES)]
        for r in range(1, NTILES):
            row_v = merge_vals_v[pl.ds(r * LANES, LANES)]
            row_i = merge_idxs_v[pl.ds(r * LANES, LANES)]
            upd = row_v > acc_v
            acc_v = jnp.where(upd, row_v, acc_v)
            acc_i = jnp.where(upd, row_i, acc_i)
        out_v[...] = jnp.where(ends_vec > starts_vec, acc_i, BIG)
        pltpu.sync_copy(out_v, out_hbm)


@functools.lru_cache(maxsize=1)
def _build():
  return pl.kernel(
    _body,
    out_type=jax.ShapeDtypeStruct((NSEG,), jnp.int32),
    mesh=plsc.VectorSubcoreMesh(
        core_axis_name="c", subcore_axis_name="s",
        num_cores=1, num_subcores=NTILES),
    scratch_types=[
        pltpu.VMEM((CHUNK,), jnp.float32),
        pltpu.VMEM((NSEG,), jnp.int32),
        pltpu.VMEM((NSEG,), jnp.int32),
        pltpu.VMEM((LANES,), jnp.float32),
        pltpu.VMEM((LANES,), jnp.int32),
        pltpu.VMEM_SHARED((NTILES * LANES,), jnp.float32),
        pltpu.VMEM_SHARED((NTILES * LANES,), jnp.int32),
        pltpu.VMEM((NTILES * LANES,), jnp.float32),
        pltpu.VMEM((NTILES * LANES,), jnp.int32),
        pltpu.VMEM((NSEG,), jnp.int32),
    ],
    compiler_params=pltpu.CompilerParams(needs_layout_passes=False),
  )


def kernel(values, prefix_sum):
    ps = prefix_sum.astype(jnp.int32)
    starts = jnp.concatenate([jnp.zeros((1,), jnp.int32), ps[:-1]])
    out = _build()(values, starts, ps)
    return out.astype(jnp.int64)
